# baseline (speedup 1.0000x reference)
import jax
import jax.numpy as jnp
from jax import lax
from jax.experimental import pallas as pl
from jax.experimental.pallas import tpu as pltpu
from jax.experimental.pallas import tpu_sc as plsc

NC = 2
NS = 16
NW = NC * NS
N_PAD = 10240
RPT = N_PAD // NS
DW = 16


def _sc_mesh():
    return plsc.VectorSubcoreMesh(
        core_axis_name="c", subcore_axis_name="s", num_cores=NC, num_subcores=NS
    )


def _sc_aggregate(table, src_p, dst_p, ept0, ept1, k, with_deg=False):
    n, d = table.shape
    nstripe = RPT // k
    n0, n1 = ept0 // k, ept1 // k
    out_type = [jax.ShapeDtypeStruct((NC, N_PAD, d), jnp.float32)]
    scratch = [
        pltpu.VMEM((k,), jnp.int32),
        pltpu.VMEM((k,), jnp.int32),
        pltpu.VMEM((k,), jnp.int32),
        pltpu.VMEM((k,), jnp.int32),
        pltpu.VMEM((k, d), jnp.float32),
        pltpu.VMEM((k, d), jnp.float32),
        pltpu.VMEM_SHARED((N_PAD, d), jnp.float32),
        pltpu.SemaphoreType.DMA,
        pltpu.SemaphoreType.DMA,
    ]
    if with_deg:
        out_type.append(jax.ShapeDtypeStruct((NC, N_PAD, d), jnp.float32))

    def body(*refs):
        if with_deg:
            (table_h, src_h, dst_h, aggp, degp,
             srcv0, srcv1, dstv0, dstv1, rows0, rows1, acc, sem0, sem1) = refs
        else:
            (table_h, src_h, dst_h, aggp,
             srcv0, srcv1, dstv0, dstv1, rows0, rows1, acc, sem0, sem1) = refs
        srcv = (srcv0, srcv1)
        dstv = (dstv0, dstv1)
        rowsv = (rows0, rows1)
        sems = (sem0, sem1)
        c = lax.axis_index("c")
        s = lax.axis_index("s")
        r0 = s * RPT
        nchunks = jnp.where(c == 0, n0, n1)
        tile_base = jnp.where(c == 0, s * ept0, NS * ept0 + s * ept1)

        def fill(buf, nrows, val):
            vec = jnp.full((16,), val, jnp.float32)

            def fr(i, carry):
                for c16 in range(d // 16):
                    buf[i, pl.ds(16 * c16, 16)] = vec
                return carry

            lax.fori_loop(0, nrows, fr, 0)

        def zero_acc():
            fill(rows0, k, 0.0)
            for t in range(nstripe):
                pltpu.sync_copy(rows0, acc.at[pl.ds(r0 + t * k, k)])

        def writeback(out_h):
            for t in range(nstripe):
                pltpu.sync_copy(acc.at[pl.ds(r0 + t * k, k)], rows0)
                pltpu.sync_copy(rows0, out_h.at[c, pl.ds(r0 + t * k, k)])

        def load_idx(j, b, with_src):
            base = tile_base + j * k
            if with_src:
                pltpu.sync_copy(src_h.at[pl.ds(base, k)], srcv[b])
            pltpu.sync_copy(dst_h.at[pl.ds(base, k)], dstv[b])

        if with_deg:
            zero_acc()
            fill(rows1, k, 1.0)
            plsc.subcore_barrier()
            for b in range(2):
                load_idx(b, b, False)
                pltpu.async_copy(rows1, acc.at[dstv[b]], sems[b], add=True)

            def dstep(i, carry):
                for b in range(2):
                    j = 2 * i + b
                    pltpu.make_async_copy(rows1, acc.at[dstv[b]], sems[b]).wait()

                    @pl.when(j + 2 < nchunks)
                    def _():
                        load_idx(j + 2, b, False)
                        pltpu.async_copy(rows1, acc.at[dstv[b]], sems[b], add=True)

                return carry

            lax.fori_loop(0, nchunks // 2, dstep, 0)
            plsc.subcore_barrier()
            writeback(degp)
            plsc.subcore_barrier()

        zero_acc()
        plsc.subcore_barrier()
        for b in range(2):
            load_idx(b, b, True)
            pltpu.async_copy(table_h.at[srcv[b]], rowsv[b], sems[b])

        def step(i, carry):
            for b in range(2):
                j = 2 * i + b
                pltpu.make_async_copy(table_h.at[srcv[b]], rowsv[b], sems[b]).wait()
                pltpu.sync_copy(rowsv[b], acc.at[dstv[b]], add=True)

                @pl.when(j + 2 < nchunks)
                def _():
                    load_idx(j + 2, b, True)
                    pltpu.async_copy(table_h.at[srcv[b]], rowsv[b], sems[b])

            return carry

        lax.fori_loop(0, nchunks // 2, step, 0)
        plsc.subcore_barrier()
        writeback(aggp)

    ins = (table, src_p, dst_p)
    return pl.kernel(body, out_type=out_type, mesh=_sc_mesh(), scratch_types=scratch)(*ins)


def _sc_pair_gather(table, idx):
    n, d = table.shape
    b = idx.shape[0]
    k = 128
    per_w = b // NW
    chunks = per_w // k

    def body(table_h, idx_h, out_h, idxv, rows, sem):
        c = lax.axis_index("c")
        s = lax.axis_index("s")
        w = c * NS + s

        def step(j, carry):
            base = w * per_w + j * k
            pltpu.sync_copy(idx_h.at[pl.ds(base, k)], idxv)
            pltpu.async_copy(table_h.at[idxv], rows, sem).wait()
            pltpu.sync_copy(rows, out_h.at[pl.ds(base, k)])
            return carry

        lax.fori_loop(0, chunks, step, 0)

    return pl.kernel(
        body,
        out_type=jax.ShapeDtypeStruct((b, d), jnp.float32),
        mesh=_sc_mesh(),
        scratch_types=[
            pltpu.VMEM((k,), jnp.int32),
            pltpu.VMEM((k, d), jnp.float32),
            pltpu.SemaphoreType.DMA,
        ],
    )(table, idx)


_BLK = 512


def _tc_layer0(h, aggp, degp, Ws, Wn, b):
    n, d = h.shape
    hdim = Ws.shape[1]
    b2 = b.reshape(1, hdim)

    def body(h_ref, a_ref, d_ref, ws_ref, wn_ref, b_ref, o_ref, inv_ref):
        a = a_ref[...]
        agg = a[0] + a[1]
        dg = d_ref[...]
        inv = 1.0 / jnp.maximum((dg[0] + dg[1])[:, :1], 1.0)
        hn = agg * inv
        acc = jnp.dot(h_ref[...], ws_ref[...], preferred_element_type=jnp.float32)
        acc = acc + jnp.dot(hn, wn_ref[...], preferred_element_type=jnp.float32)
        o_ref[...] = jnp.maximum(acc + b_ref[...], 0.0)
        inv_ref[...] = jnp.broadcast_to(inv, (inv.shape[0], DW))

    return pl.pallas_call(
        body,
        grid=(pl.cdiv(N_PAD, _BLK),),
        in_specs=[
            pl.BlockSpec((_BLK, d), lambda i: (i, 0)),
            pl.BlockSpec((NC, _BLK, d), lambda i: (0, i, 0)),
            pl.BlockSpec((NC, _BLK, d), lambda i: (0, i, 0)),
            pl.BlockSpec((d, hdim), lambda i: (0, 0)),
            pl.BlockSpec((d, hdim), lambda i: (0, 0)),
            pl.BlockSpec((1, hdim), lambda i: (0, 0)),
        ],
        out_specs=[
            pl.BlockSpec((_BLK, hdim), lambda i: (i, 0)),
            pl.BlockSpec((_BLK, DW), lambda i: (i, 0)),
        ],
        out_shape=[
            jax.ShapeDtypeStruct((n, hdim), jnp.float32),
            jax.ShapeDtypeStruct((N_PAD, DW), jnp.float32),
        ],
    )(h, aggp, degp, Ws, Wn, b2)


def _tc_layer1(h, aggp, inv_deg, Ws, Wn, b):
    n, d = h.shape
    hdim = Ws.shape[1]
    b2 = b.reshape(1, hdim)

    def body(h_ref, a_ref, inv_ref, ws_ref, wn_ref, b_ref, o_ref):
        a = a_ref[...]
        agg = a[0] + a[1]
        inv = inv_ref[...][:, :1]
        hn = agg * inv
        acc = jnp.dot(h_ref[...], ws_ref[...], preferred_element_type=jnp.float32)
        acc = acc + jnp.dot(hn, wn_ref[...], preferred_element_type=jnp.float32)
        o_ref[...] = jnp.maximum(acc + b_ref[...], 0.0)

    return pl.pallas_call(
        body,
        grid=(pl.cdiv(n, _BLK),),
        in_specs=[
            pl.BlockSpec((_BLK, d), lambda i: (i, 0)),
            pl.BlockSpec((NC, _BLK, d), lambda i: (0, i, 0)),
            pl.BlockSpec((_BLK, DW), lambda i: (i, 0)),
            pl.BlockSpec((d, hdim), lambda i: (0, 0)),
            pl.BlockSpec((d, hdim), lambda i: (0, 0)),
            pl.BlockSpec((1, hdim), lambda i: (0, 0)),
        ],
        out_specs=pl.BlockSpec((_BLK, hdim), lambda i: (i, 0)),
        out_shape=jax.ShapeDtypeStruct((n, hdim), jnp.float32),
    )(h, aggp, inv_deg, Ws, Wn, b2)


def _tc_head(hx, p, W1a, W1b, W1c, bl1, W2p, bl2p):
    hdim = W1a.shape[0]
    cp = W2p.shape[1]
    nblk = p // _BLK
    bl1_2 = bl1.reshape(1, hdim)

    def body(h1_ref, h2_ref, wa_ref, wb_ref, wc_ref, b1_ref, w2_ref, b2_ref, o_ref):
        h1 = h1_ref[...]
        h2 = h2_ref[...]
        z = jnp.dot(h1, wa_ref[...], preferred_element_type=jnp.float32)
        z = z + jnp.dot(h2, wb_ref[...], preferred_element_type=jnp.float32)
        z = z + jnp.dot(jnp.abs(h1 - h2), wc_ref[...], preferred_element_type=jnp.float32)
        z = jnp.maximum(z + b1_ref[...], 0.0)
        o_ref[...] = jnp.dot(z, w2_ref[...], preferred_element_type=jnp.float32) + b2_ref[...]

    return pl.pallas_call(
        body,
        grid=(nblk,),
        in_specs=[
            pl.BlockSpec((_BLK, hdim), lambda i: (i, 0)),
            pl.BlockSpec((_BLK, hdim), lambda i: (i + nblk, 0)),
            pl.BlockSpec((hdim, hdim), lambda i: (0, 0)),
            pl.BlockSpec((hdim, hdim), lambda i: (0, 0)),
            pl.BlockSpec((hdim, hdim), lambda i: (0, 0)),
            pl.BlockSpec((1, hdim), lambda i: (0, 0)),
            pl.BlockSpec((hdim, cp), lambda i: (0, 0)),
            pl.BlockSpec((1, cp), lambda i: (0, 0)),
        ],
        out_specs=pl.BlockSpec((_BLK, cp), lambda i: (i, 0)),
        out_shape=jax.ShapeDtypeStruct((p, cp), jnp.float32),
    )(hx, hx, W1a, W1b, W1c, bl1_2, W2p, bl2p.reshape(1, cp))


def kernel(h, edge_index, x1, x2, Ws0, Wn0, b0, Ws1, Wn1, b1, W1, bl1, W2, bl2):
    n, d = h.shape
    e = edge_index.shape[1]
    p = x1.shape[0]
    hdim = Ws0.shape[1]
    c_out = W2.shape[1]

    def _split(f0):
        ept0 = max(256, int(round(e * f0 / NS / 256)) * 256)
        ept1 = (e - NS * ept0 + NS * 256 - 1) // (NS * 256) * 256
        return ept0, ept1

    ept0a, ept1a = _split(0.26)
    ept0b, ept1b = _split(0.26)
    def _pad_edges(pad):
        src_p = jnp.concatenate([edge_index[0], jnp.zeros((pad,), jnp.int32)])
        dst_p = jnp.concatenate(
            [edge_index[1], n + jnp.arange(pad, dtype=jnp.int32) % (N_PAD - n)]
        )
        return src_p, dst_p

    src_pa, dst_pa = _pad_edges(NS * (ept0a + ept1a) - e)
    src_pb, dst_pb = _pad_edges(NS * (ept0b + ept1b) - e)

    aggp0, degp = _sc_aggregate(h, src_pa, dst_pa, ept0a, ept1a, k=128, with_deg=True)
    h1, inv_deg = _tc_layer0(h, aggp0, degp, Ws0, Wn0, b0)
    (aggp1,) = _sc_aggregate(h1, src_pb, dst_pb, ept0b, ept1b, k=128)
    h2 = _tc_layer1(h1, aggp1, inv_deg, Ws1, Wn1, b1)

    hx = _sc_pair_gather(h2, jnp.concatenate([x1, x2]))

    W1a, W1b, W1c = W1[:hdim], W1[hdim : 2 * hdim], W1[2 * hdim :]
    cp = 8
    W2p = jnp.pad(W2, ((0, 0), (0, cp - c_out)))
    bl2p = jnp.pad(bl2, (0, cp - c_out))
    out = _tc_head(hx, p, W1a, W1b, W1c, bl1, W2p, bl2p)
    return out[:, :c_out]


# speedup vs baseline: 1.0453x; 1.0453x over previous
import jax
import jax.numpy as jnp
from jax import lax
from jax.experimental import pallas as pl
from jax.experimental.pallas import tpu as pltpu
from jax.experimental.pallas import tpu_sc as plsc

NC = 2
NS = 16
NW = NC * NS
N_PAD = 10240
RPT = N_PAD // NS
DW = 16


def _sc_mesh():
    return plsc.VectorSubcoreMesh(
        core_axis_name="c", subcore_axis_name="s", num_cores=NC, num_subcores=NS
    )


def _sc_aggregate(table, src_p, dst_p, ept0, ept1, k, with_deg=False):
    n, d = table.shape
    nstripe = RPT // k
    n0, n1 = ept0 // k, ept1 // k
    out_type = [jax.ShapeDtypeStruct((NC, N_PAD, d), jnp.float32)]
    scratch = [
        pltpu.VMEM((k,), jnp.int32),
        pltpu.VMEM((k,), jnp.int32),
        pltpu.VMEM((k,), jnp.int32),
        pltpu.VMEM((k,), jnp.int32),
        pltpu.VMEM((k, d), jnp.float32),
        pltpu.VMEM((k, d), jnp.float32),
        pltpu.VMEM_SHARED((N_PAD, d), jnp.float32),
        pltpu.SemaphoreType.DMA,
        pltpu.SemaphoreType.DMA,
    ]
    if with_deg:
        out_type.append(jax.ShapeDtypeStruct((NC, N_PAD, d), jnp.float32))

    def body(*refs):
        if with_deg:
            (table_h, src_h, dst_h, aggp, degp,
             srcv0, srcv1, dstv0, dstv1, rows0, rows1, acc, sem0, sem1) = refs
        else:
            (table_h, src_h, dst_h, aggp,
             srcv0, srcv1, dstv0, dstv1, rows0, rows1, acc, sem0, sem1) = refs
        srcv = (srcv0, srcv1)
        dstv = (dstv0, dstv1)
        rowsv = (rows0, rows1)
        sems = (sem0, sem1)
        c = lax.axis_index("c")
        s = lax.axis_index("s")
        r0 = s * RPT
        nchunks = jnp.where(c == 0, n0, n1)
        tile_base = jnp.where(c == 0, s * ept0, NS * ept0 + s * ept1)

        def fill(buf, nrows, val):
            vec = jnp.full((16,), val, jnp.float32)

            def fr(i, carry):
                for c16 in range(d // 16):
                    buf[i, pl.ds(16 * c16, 16)] = vec
                return carry

            lax.fori_loop(0, nrows, fr, 0)

        def zero_acc():
            fill(rows0, k, 0.0)
            for t in range(nstripe):
                pltpu.sync_copy(rows0, acc.at[pl.ds(r0 + t * k, k)])

        def writeback(out_h):
            for t in range(nstripe):
                pltpu.sync_copy(acc.at[pl.ds(r0 + t * k, k)], rows0)
                pltpu.sync_copy(rows0, out_h.at[c, pl.ds(r0 + t * k, k)])

        def load_idx(j, b, with_src):
            base = tile_base + j * k
            if with_src:
                pltpu.sync_copy(src_h.at[pl.ds(base, k)], srcv[b])
            pltpu.sync_copy(dst_h.at[pl.ds(base, k)], dstv[b])

        if with_deg:
            zero_acc()
            fill(rows1, k, 1.0)
            plsc.subcore_barrier()
            for b in range(2):
                load_idx(b, b, False)
                pltpu.async_copy(rows1, acc.at[dstv[b]], sems[b], add=True)

            def dstep(i, carry):
                for b in range(2):
                    j = 2 * i + b
                    pltpu.make_async_copy(rows1, acc.at[dstv[b]], sems[b]).wait()

                    @pl.when(j + 2 < nchunks)
                    def _():
                        load_idx(j + 2, b, False)
                        pltpu.async_copy(rows1, acc.at[dstv[b]], sems[b], add=True)

                return carry

            lax.fori_loop(0, nchunks // 2, dstep, 0)
            plsc.subcore_barrier()
            writeback(degp)
            plsc.subcore_barrier()

        zero_acc()
        plsc.subcore_barrier()
        for b in range(2):
            load_idx(b, b, True)
            pltpu.async_copy(table_h.at[srcv[b]], rowsv[b], sems[b])

        def step(i, carry):
            for b in range(2):
                j = 2 * i + b
                pltpu.make_async_copy(table_h.at[srcv[b]], rowsv[b], sems[b]).wait()
                pltpu.sync_copy(rowsv[b], acc.at[dstv[b]], add=True)

                @pl.when(j + 2 < nchunks)
                def _():
                    load_idx(j + 2, b, True)
                    pltpu.async_copy(table_h.at[srcv[b]], rowsv[b], sems[b])

            return carry

        lax.fori_loop(0, nchunks // 2, step, 0)
        plsc.subcore_barrier()
        writeback(aggp)

    ins = (table, src_p, dst_p)
    return pl.kernel(body, out_type=out_type, mesh=_sc_mesh(), scratch_types=scratch)(*ins)


def _sc_pair_gather(table, idx):
    n, d = table.shape
    b = idx.shape[0]
    k = 128
    per_w = b // NW
    chunks = per_w // k

    def body(table_h, idx_h, out_h, idxv, rows, sem):
        c = lax.axis_index("c")
        s = lax.axis_index("s")
        w = c * NS + s

        def step(j, carry):
            base = w * per_w + j * k
            pltpu.sync_copy(idx_h.at[pl.ds(base, k)], idxv)
            pltpu.async_copy(table_h.at[idxv], rows, sem).wait()
            pltpu.sync_copy(rows, out_h.at[pl.ds(base, k)])
            return carry

        lax.fori_loop(0, chunks, step, 0)

    return pl.kernel(
        body,
        out_type=jax.ShapeDtypeStruct((b, d), jnp.float32),
        mesh=_sc_mesh(),
        scratch_types=[
            pltpu.VMEM((k,), jnp.int32),
            pltpu.VMEM((k, d), jnp.float32),
            pltpu.SemaphoreType.DMA,
        ],
    )(table, idx)


_BLK = 512


def _tc_layer0(h, aggp, degp, Ws, Wn, b):
    n, d = h.shape
    hdim = Ws.shape[1]
    b2 = b.reshape(1, hdim)

    def body(h_ref, a_ref, d_ref, ws_ref, wn_ref, b_ref, o_ref, inv_ref):
        a = a_ref[...]
        agg = a[0] + a[1]
        dg = d_ref[...]
        inv = 1.0 / jnp.maximum((dg[0] + dg[1])[:, :1], 1.0)
        hn = agg * inv
        acc = jnp.dot(h_ref[...], ws_ref[...], preferred_element_type=jnp.float32)
        acc = acc + jnp.dot(hn, wn_ref[...], preferred_element_type=jnp.float32)
        o_ref[...] = jnp.maximum(acc + b_ref[...], 0.0)
        inv_ref[...] = jnp.broadcast_to(inv, (inv.shape[0], DW))

    return pl.pallas_call(
        body,
        grid=(pl.cdiv(N_PAD, _BLK),),
        in_specs=[
            pl.BlockSpec((_BLK, d), lambda i: (i, 0)),
            pl.BlockSpec((NC, _BLK, d), lambda i: (0, i, 0)),
            pl.BlockSpec((NC, _BLK, d), lambda i: (0, i, 0)),
            pl.BlockSpec((d, hdim), lambda i: (0, 0)),
            pl.BlockSpec((d, hdim), lambda i: (0, 0)),
            pl.BlockSpec((1, hdim), lambda i: (0, 0)),
        ],
        out_specs=[
            pl.BlockSpec((_BLK, hdim), lambda i: (i, 0)),
            pl.BlockSpec((_BLK, DW), lambda i: (i, 0)),
        ],
        out_shape=[
            jax.ShapeDtypeStruct((n, hdim), jnp.float32),
            jax.ShapeDtypeStruct((N_PAD, DW), jnp.float32),
        ],
    )(h, aggp, degp, Ws, Wn, b2)


def _tc_layer1(h, aggp, inv_deg, Ws, Wn, b):
    n, d = h.shape
    hdim = Ws.shape[1]
    b2 = b.reshape(1, hdim)

    def body(h_ref, a_ref, inv_ref, ws_ref, wn_ref, b_ref, o_ref):
        a = a_ref[...]
        agg = a[0] + a[1]
        inv = inv_ref[...][:, :1]
        hn = agg * inv
        acc = jnp.dot(h_ref[...], ws_ref[...], preferred_element_type=jnp.float32)
        acc = acc + jnp.dot(hn, wn_ref[...], preferred_element_type=jnp.float32)
        o_ref[...] = jnp.maximum(acc + b_ref[...], 0.0)

    return pl.pallas_call(
        body,
        grid=(pl.cdiv(n, _BLK),),
        in_specs=[
            pl.BlockSpec((_BLK, d), lambda i: (i, 0)),
            pl.BlockSpec((NC, _BLK, d), lambda i: (0, i, 0)),
            pl.BlockSpec((_BLK, DW), lambda i: (i, 0)),
            pl.BlockSpec((d, hdim), lambda i: (0, 0)),
            pl.BlockSpec((d, hdim), lambda i: (0, 0)),
            pl.BlockSpec((1, hdim), lambda i: (0, 0)),
        ],
        out_specs=pl.BlockSpec((_BLK, hdim), lambda i: (i, 0)),
        out_shape=jax.ShapeDtypeStruct((n, hdim), jnp.float32),
    )(h, aggp, inv_deg, Ws, Wn, b2)


def _tc_head(hx, p, W1a, W1b, W1c, bl1, W2p, bl2p):
    hdim = W1a.shape[0]
    cp = W2p.shape[1]
    nblk = p // _BLK
    bl1_2 = bl1.reshape(1, hdim)

    def body(h1_ref, h2_ref, wa_ref, wb_ref, wc_ref, b1_ref, w2_ref, b2_ref, o_ref):
        h1 = h1_ref[...]
        h2 = h2_ref[...]
        z = jnp.dot(h1, wa_ref[...], preferred_element_type=jnp.float32)
        z = z + jnp.dot(h2, wb_ref[...], preferred_element_type=jnp.float32)
        z = z + jnp.dot(jnp.abs(h1 - h2), wc_ref[...], preferred_element_type=jnp.float32)
        z = jnp.maximum(z + b1_ref[...], 0.0)
        o_ref[...] = jnp.dot(z, w2_ref[...], preferred_element_type=jnp.float32) + b2_ref[...]

    return pl.pallas_call(
        body,
        grid=(nblk,),
        in_specs=[
            pl.BlockSpec((_BLK, hdim), lambda i: (i, 0)),
            pl.BlockSpec((_BLK, hdim), lambda i: (i + nblk, 0)),
            pl.BlockSpec((hdim, hdim), lambda i: (0, 0)),
            pl.BlockSpec((hdim, hdim), lambda i: (0, 0)),
            pl.BlockSpec((hdim, hdim), lambda i: (0, 0)),
            pl.BlockSpec((1, hdim), lambda i: (0, 0)),
            pl.BlockSpec((hdim, cp), lambda i: (0, 0)),
            pl.BlockSpec((1, cp), lambda i: (0, 0)),
        ],
        out_specs=pl.BlockSpec((_BLK, cp), lambda i: (i, 0)),
        out_shape=jax.ShapeDtypeStruct((p, cp), jnp.float32),
    )(hx, hx, W1a, W1b, W1c, bl1_2, W2p, bl2p.reshape(1, cp))


def kernel(h, edge_index, x1, x2, Ws0, Wn0, b0, Ws1, Wn1, b1, W1, bl1, W2, bl2):
    n, d = h.shape
    e = edge_index.shape[1]
    p = x1.shape[0]
    hdim = Ws0.shape[1]
    c_out = W2.shape[1]

    def _split(f0):
        ept0 = max(256, int(round(e * f0 / NS / 256)) * 256)
        ept1 = (e - NS * ept0 + NS * 256 - 1) // (NS * 256) * 256
        return ept0, ept1

    ept0a, ept1a = _split(0.32)
    ept0b, ept1b = _split(0.32)
    def _pad_edges(pad):
        src_p = jnp.concatenate([edge_index[0], jnp.zeros((pad,), jnp.int32)])
        dst_p = jnp.concatenate(
            [edge_index[1], n + jnp.arange(pad, dtype=jnp.int32) % (N_PAD - n)]
        )
        return src_p, dst_p

    src_pa, dst_pa = _pad_edges(NS * (ept0a + ept1a) - e)
    src_pb, dst_pb = _pad_edges(NS * (ept0b + ept1b) - e)

    aggp0, degp = _sc_aggregate(h, src_pa, dst_pa, ept0a, ept1a, k=128, with_deg=True)
    h1, inv_deg = _tc_layer0(h, aggp0, degp, Ws0, Wn0, b0)
    (aggp1,) = _sc_aggregate(h1, src_pb, dst_pb, ept0b, ept1b, k=128)
    h2 = _tc_layer1(h1, aggp1, inv_deg, Ws1, Wn1, b1)

    hx = _sc_pair_gather(h2, jnp.concatenate([x1, x2]))

    W1a, W1b, W1c = W1[:hdim], W1[hdim : 2 * hdim], W1[2 * hdim :]
    cp = 8
    W2p = jnp.pad(W2, ((0, 0), (0, cp - c_out)))
    bl2p = jnp.pad(bl2, (0, cp - c_out))
    out = _tc_head(hx, p, W1a, W1b, W1c, bl1, W2p, bl2p)
    return out[:, :c_out]


# speedup vs baseline: 1.0634x; 1.0173x over previous
import jax
import jax.numpy as jnp
from jax import lax
from jax.experimental import pallas as pl
from jax.experimental.pallas import tpu as pltpu
from jax.experimental.pallas import tpu_sc as plsc

NC = 2
NS = 16
NW = NC * NS
N_PAD = 10240
RPT = N_PAD // NS
DW = 16


def _sc_mesh():
    return plsc.VectorSubcoreMesh(
        core_axis_name="c", subcore_axis_name="s", num_cores=NC, num_subcores=NS
    )


def _sc_aggregate(table, src_p, dst_p, ept0, ept1, k, with_deg=False):
    n, d = table.shape
    nstripe = RPT // k
    n0, n1 = ept0 // k, ept1 // k
    out_type = [jax.ShapeDtypeStruct((NC, N_PAD, d), jnp.float32)]
    scratch = [
        pltpu.VMEM((k,), jnp.int32),
        pltpu.VMEM((k,), jnp.int32),
        pltpu.VMEM((k,), jnp.int32),
        pltpu.VMEM((k,), jnp.int32),
        pltpu.VMEM((k, d), jnp.float32),
        pltpu.VMEM((k, d), jnp.float32),
        pltpu.VMEM_SHARED((N_PAD, d), jnp.float32),
        pltpu.SemaphoreType.DMA,
        pltpu.SemaphoreType.DMA,
    ]
    if with_deg:
        out_type.append(jax.ShapeDtypeStruct((NC, N_PAD, d), jnp.float32))

    def body(*refs):
        if with_deg:
            (table_h, src_h, dst_h, aggp, degp,
             srcv0, srcv1, dstv0, dstv1, rows0, rows1, acc, sem0, sem1) = refs
        else:
            (table_h, src_h, dst_h, aggp,
             srcv0, srcv1, dstv0, dstv1, rows0, rows1, acc, sem0, sem1) = refs
        srcv = (srcv0, srcv1)
        dstv = (dstv0, dstv1)
        rowsv = (rows0, rows1)
        sems = (sem0, sem1)
        c = lax.axis_index("c")
        s = lax.axis_index("s")
        r0 = s * RPT
        nchunks = jnp.where(c == 0, n0, n1)
        tile_base = jnp.where(c == 0, s * ept0, NS * ept0 + s * ept1)

        def fill(buf, nrows, val):
            vec = jnp.full((16,), val, jnp.float32)

            def fr(i, carry):
                for c16 in range(d // 16):
                    buf[i, pl.ds(16 * c16, 16)] = vec
                return carry

            lax.fori_loop(0, nrows, fr, 0)

        def zero_acc():
            fill(rows0, k, 0.0)
            for t in range(nstripe):
                pltpu.sync_copy(rows0, acc.at[pl.ds(r0 + t * k, k)])

        def writeback(out_h):
            for t in range(nstripe):
                pltpu.sync_copy(acc.at[pl.ds(r0 + t * k, k)], rows0)
                pltpu.sync_copy(rows0, out_h.at[c, pl.ds(r0 + t * k, k)])

        def load_idx(j, b, with_src):
            base = tile_base + j * k
            if with_src:
                pltpu.sync_copy(src_h.at[pl.ds(base, k)], srcv[b])
            pltpu.sync_copy(dst_h.at[pl.ds(base, k)], dstv[b])

        if with_deg:
            zero_acc()
            fill(rows1, k, 1.0)
            plsc.subcore_barrier()
            for b in range(2):
                load_idx(b, b, False)
                pltpu.async_copy(rows1, acc.at[dstv[b]], sems[b], add=True)

            def dstep(i, carry):
                for b in range(2):
                    j = 2 * i + b
                    pltpu.make_async_copy(rows1, acc.at[dstv[b]], sems[b]).wait()

                    @pl.when(j + 2 < nchunks)
                    def _():
                        load_idx(j + 2, b, False)
                        pltpu.async_copy(rows1, acc.at[dstv[b]], sems[b], add=True)

                return carry

            lax.fori_loop(0, nchunks // 2, dstep, 0)
            plsc.subcore_barrier()
            writeback(degp)
            plsc.subcore_barrier()

        zero_acc()
        plsc.subcore_barrier()
        for b in range(2):
            load_idx(b, b, True)
            pltpu.async_copy(table_h.at[srcv[b]], rowsv[b], sems[b])

        def step(i, carry):
            for b in range(2):
                j = 2 * i + b
                pltpu.make_async_copy(table_h.at[srcv[b]], rowsv[b], sems[b]).wait()
                pltpu.sync_copy(rowsv[b], acc.at[dstv[b]], add=True)

                @pl.when(j + 2 < nchunks)
                def _():
                    load_idx(j + 2, b, True)
                    pltpu.async_copy(table_h.at[srcv[b]], rowsv[b], sems[b])

            return carry

        lax.fori_loop(0, nchunks // 2, step, 0)
        plsc.subcore_barrier()
        writeback(aggp)

    ins = (table, src_p, dst_p)
    return pl.kernel(body, out_type=out_type, mesh=_sc_mesh(), scratch_types=scratch)(*ins)


def _sc_pair_gather(table, idx):
    n, d = table.shape
    b = idx.shape[0]
    k = 128
    per_w = b // NW
    chunks = per_w // k

    def body(table_h, idx_h, out_h, idxv, rows, sem):
        c = lax.axis_index("c")
        s = lax.axis_index("s")
        w = c * NS + s

        def step(j, carry):
            base = w * per_w + j * k
            pltpu.sync_copy(idx_h.at[pl.ds(base, k)], idxv)
            pltpu.async_copy(table_h.at[idxv], rows, sem).wait()
            pltpu.sync_copy(rows, out_h.at[pl.ds(base, k)])
            return carry

        lax.fori_loop(0, chunks, step, 0)

    return pl.kernel(
        body,
        out_type=jax.ShapeDtypeStruct((b, d), jnp.float32),
        mesh=_sc_mesh(),
        scratch_types=[
            pltpu.VMEM((k,), jnp.int32),
            pltpu.VMEM((k, d), jnp.float32),
            pltpu.SemaphoreType.DMA,
        ],
    )(table, idx)


_BLK = 512


def _tc_layer0(h, aggp, degp, Ws, Wn, b):
    n, d = h.shape
    hdim = Ws.shape[1]
    b2 = b.reshape(1, hdim)

    def body(h_ref, a_ref, d_ref, ws_ref, wn_ref, b_ref, o_ref, inv_ref):
        a = a_ref[...]
        agg = a[0] + a[1]
        dg = d_ref[...]
        inv = 1.0 / jnp.maximum((dg[0] + dg[1])[:, :1], 1.0)
        hn = agg * inv
        acc = jnp.dot(h_ref[...], ws_ref[...], preferred_element_type=jnp.float32)
        acc = acc + jnp.dot(hn, wn_ref[...], preferred_element_type=jnp.float32)
        o_ref[...] = jnp.maximum(acc + b_ref[...], 0.0)
        inv_ref[...] = jnp.broadcast_to(inv, (inv.shape[0], DW))

    return pl.pallas_call(
        body,
        grid=(pl.cdiv(N_PAD, _BLK),),
        in_specs=[
            pl.BlockSpec((_BLK, d), lambda i: (i, 0)),
            pl.BlockSpec((NC, _BLK, d), lambda i: (0, i, 0)),
            pl.BlockSpec((NC, _BLK, d), lambda i: (0, i, 0)),
            pl.BlockSpec((d, hdim), lambda i: (0, 0)),
            pl.BlockSpec((d, hdim), lambda i: (0, 0)),
            pl.BlockSpec((1, hdim), lambda i: (0, 0)),
        ],
        out_specs=[
            pl.BlockSpec((_BLK, hdim), lambda i: (i, 0)),
            pl.BlockSpec((_BLK, DW), lambda i: (i, 0)),
        ],
        out_shape=[
            jax.ShapeDtypeStruct((n, hdim), jnp.float32),
            jax.ShapeDtypeStruct((N_PAD, DW), jnp.float32),
        ],
    )(h, aggp, degp, Ws, Wn, b2)


def _tc_layer1(h, aggp, inv_deg, Ws, Wn, b):
    n, d = h.shape
    hdim = Ws.shape[1]
    b2 = b.reshape(1, hdim)

    def body(h_ref, a_ref, inv_ref, ws_ref, wn_ref, b_ref, o_ref):
        a = a_ref[...]
        agg = a[0] + a[1]
        inv = inv_ref[...][:, :1]
        hn = agg * inv
        acc = jnp.dot(h_ref[...], ws_ref[...], preferred_element_type=jnp.float32)
        acc = acc + jnp.dot(hn, wn_ref[...], preferred_element_type=jnp.float32)
        o_ref[...] = jnp.maximum(acc + b_ref[...], 0.0)

    return pl.pallas_call(
        body,
        grid=(pl.cdiv(n, _BLK),),
        in_specs=[
            pl.BlockSpec((_BLK, d), lambda i: (i, 0)),
            pl.BlockSpec((NC, _BLK, d), lambda i: (0, i, 0)),
            pl.BlockSpec((_BLK, DW), lambda i: (i, 0)),
            pl.BlockSpec((d, hdim), lambda i: (0, 0)),
            pl.BlockSpec((d, hdim), lambda i: (0, 0)),
            pl.BlockSpec((1, hdim), lambda i: (0, 0)),
        ],
        out_specs=pl.BlockSpec((_BLK, hdim), lambda i: (i, 0)),
        out_shape=jax.ShapeDtypeStruct((n, hdim), jnp.float32),
    )(h, aggp, inv_deg, Ws, Wn, b2)


def _tc_head(hx, p, W1a, W1b, W1c, bl1, W2p, bl2p):
    hdim = W1a.shape[0]
    cp = W2p.shape[1]
    nblk = p // _BLK
    bl1_2 = bl1.reshape(1, hdim)

    def body(h1_ref, h2_ref, wa_ref, wb_ref, wc_ref, b1_ref, w2_ref, b2_ref, o_ref):
        h1 = h1_ref[...]
        h2 = h2_ref[...]
        z = jnp.dot(h1, wa_ref[...], preferred_element_type=jnp.float32)
        z = z + jnp.dot(h2, wb_ref[...], preferred_element_type=jnp.float32)
        z = z + jnp.dot(jnp.abs(h1 - h2), wc_ref[...], preferred_element_type=jnp.float32)
        z = jnp.maximum(z + b1_ref[...], 0.0)
        o_ref[...] = jnp.dot(z, w2_ref[...], preferred_element_type=jnp.float32) + b2_ref[...]

    return pl.pallas_call(
        body,
        grid=(nblk,),
        in_specs=[
            pl.BlockSpec((_BLK, hdim), lambda i: (i, 0)),
            pl.BlockSpec((_BLK, hdim), lambda i: (i + nblk, 0)),
            pl.BlockSpec((hdim, hdim), lambda i: (0, 0)),
            pl.BlockSpec((hdim, hdim), lambda i: (0, 0)),
            pl.BlockSpec((hdim, hdim), lambda i: (0, 0)),
            pl.BlockSpec((1, hdim), lambda i: (0, 0)),
            pl.BlockSpec((hdim, cp), lambda i: (0, 0)),
            pl.BlockSpec((1, cp), lambda i: (0, 0)),
        ],
        out_specs=pl.BlockSpec((_BLK, cp), lambda i: (i, 0)),
        out_shape=jax.ShapeDtypeStruct((p, cp), jnp.float32),
    )(hx, hx, W1a, W1b, W1c, bl1_2, W2p, bl2p.reshape(1, cp))


def kernel(h, edge_index, x1, x2, Ws0, Wn0, b0, Ws1, Wn1, b1, W1, bl1, W2, bl2):
    n, d = h.shape
    e = edge_index.shape[1]
    p = x1.shape[0]
    hdim = Ws0.shape[1]
    c_out = W2.shape[1]

    def _split(f0):
        ept0 = max(256, int(round(e * f0 / NS / 256)) * 256)
        ept1 = (e - NS * ept0 + NS * 256 - 1) // (NS * 256) * 256
        return ept0, ept1

    ept0a, ept1a = _split(0.35)
    ept0b, ept1b = _split(0.35)
    def _pad_edges(pad):
        src_p = jnp.concatenate([edge_index[0], jnp.zeros((pad,), jnp.int32)])
        dst_p = jnp.concatenate(
            [edge_index[1], n + jnp.arange(pad, dtype=jnp.int32) % (N_PAD - n)]
        )
        return src_p, dst_p

    src_pa, dst_pa = _pad_edges(NS * (ept0a + ept1a) - e)
    src_pb, dst_pb = _pad_edges(NS * (ept0b + ept1b) - e)

    aggp0, degp = _sc_aggregate(h, src_pa, dst_pa, ept0a, ept1a, k=128, with_deg=True)
    h1, inv_deg = _tc_layer0(h, aggp0, degp, Ws0, Wn0, b0)
    (aggp1,) = _sc_aggregate(h1, src_pb, dst_pb, ept0b, ept1b, k=128)
    h2 = _tc_layer1(h1, aggp1, inv_deg, Ws1, Wn1, b1)

    hx = _sc_pair_gather(h2, jnp.concatenate([x1, x2]))

    W1a, W1b, W1c = W1[:hdim], W1[hdim : 2 * hdim], W1[2 * hdim :]
    cp = 8
    W2p = jnp.pad(W2, ((0, 0), (0, cp - c_out)))
    bl2p = jnp.pad(bl2, (0, cp - c_out))
    out = _tc_head(hx, p, W1a, W1b, W1c, bl1, W2p, bl2p)
    return out[:, :c_out]
